# Initial kernel scaffold; baseline (speedup 1.0000x reference)
import jax
import jax.numpy as jnp
from jax.experimental import pallas as pl


def kernel(x, W):
    raise NotImplementedError("write your pallas kernel here")


# speedup vs baseline: 1.2477x; 1.2477x over previous
import functools
import jax
import jax.numpy as jnp
from jax.experimental import pallas as pl

_HIDDEN = 4096
_EXPERTS = 64
_K = 8
_BT = 1024


def _router_body(x_ref, w_ref, scores_ref, weights_ref, indices_ref):
    x = x_ref[...]
    w = w_ref[...]
    logits = jax.lax.dot_general(
        x, w, (((1,), (1,)), ((), ())),
        preferred_element_type=jnp.float32,
    )
    m = jnp.max(logits, axis=1, keepdims=True)
    e = jnp.exp(logits - m)
    s = e / jnp.sum(e, axis=1, keepdims=True)
    scores_ref[...] = s

    iota = jax.lax.broadcasted_iota(jnp.int32, (_BT, _EXPERTS), 1)
    work = s
    ws = []
    ids = []
    for _ in range(_K):
        cur = jnp.max(work, axis=1, keepdims=True)
        cand = jnp.where(work == cur, iota, _EXPERTS)
        idx = jnp.min(cand, axis=1, keepdims=True)
        ws.append(cur)
        ids.append(idx)
        work = jnp.where(iota == idx, -1.0, work)
    weights_ref[...] = jnp.concatenate(ws, axis=1)
    indices_ref[...] = jnp.concatenate(ids, axis=1)


@jax.jit
def kernel(x, W):
    tokens = x.shape[0]
    grid = (tokens // _BT,)
    return pl.pallas_call(
        _router_body,
        grid=grid,
        in_specs=[
            pl.BlockSpec((_BT, _HIDDEN), lambda i: (i, 0)),
            pl.BlockSpec((_EXPERTS, _HIDDEN), lambda i: (0, 0)),
        ],
        out_specs=[
            pl.BlockSpec((_BT, _EXPERTS), lambda i: (i, 0)),
            pl.BlockSpec((_BT, _K), lambda i: (i, 0)),
            pl.BlockSpec((_BT, _K), lambda i: (i, 0)),
        ],
        out_shape=[
            jax.ShapeDtypeStruct((tokens, _EXPERTS), jnp.float32),
            jax.ShapeDtypeStruct((tokens, _K), jnp.float32),
            jax.ShapeDtypeStruct((tokens, _K), jnp.int32),
        ],
    )(x, W)
